# baseline (speedup 1.0000x reference)
import jax
import jax.numpy as jnp
from jax import lax

_SPLAT_DNUMS = lax.GatherDimensionNumbers(
    offset_dims=(), collapsed_slice_dims=(0,), start_index_map=(0,))


def _splat(vec, j):
    idx = jnp.full((16, 1), j, jnp.int32)
    return lax.gather(vec, idx, _SPLAT_DNUMS, (1,),
                      mode=lax.GatherScatterMode.PROMISE_IN_BOUNDS)
from jax.experimental import pallas as pl
from jax.experimental.pallas import tpu as pltpu
from jax.experimental.pallas import tpu_sc as plsc

N = 10000
E = 320000
R = 7
D = 128
HD = D // 2

NC = 2
NS = 16
EPT = E // NS
CH = 128
NCH = 159
EPTP = NCH * CH
NBUF = 3
RC = EPTP // 6
RPW = N // NS
L = 16


def _sc_body(y_hbm, src_hbm, rel_hbm, dst_hbm, w_hbm, z_hbm, out_hbm,
             srcv, relv, dstv, wv, rows0, rows1, rows2, acc,
             g0, g1, g2, s0, s1, s2):
    c = lax.axis_index("c")
    s = lax.axis_index("s")
    rows = (rows0, rows1, rows2)
    gsem = (g0, g1, g2)
    ssem = (s0, s1, s2)

    pltpu.sync_copy(z_hbm, acc.at[pl.ds(s * RPW, RPW)])

    pltpu.sync_copy(src_hbm.at[s], srcv)
    pltpu.sync_copy(dst_hbm.at[s], dstv)
    pltpu.sync_copy(w_hbm.at[s], wv)

    for seg in range(EPTP // RC):
        pltpu.sync_copy(rel_hbm.at[s].at[pl.ds(seg * RC, RC)], relv)

        def _gidx(t, _):
            gl = pl.ds(seg * RC + t * L, L)
            srcv[gl] = srcv[gl] * R + relv[pl.ds(t * L, L)]
            return 0
        lax.fori_loop(0, RC // L, _gidx, 0)

    plsc.subcore_barrier()

    def _edge_loop(y_ref):
        def _gather(ci, b):
            pltpu.async_copy(y_ref.at[srcv.at[pl.ds(ci * CH, CH)]],
                             rows[b], gsem[b])

        def _gather_wait(ci, b):
            pltpu.make_async_copy(y_ref.at[srcv.at[pl.ds(ci * CH, CH)]],
                                  rows[b], gsem[b]).wait()

        def _scatter_wait(ci, b):
            pltpu.make_async_copy(rows[b], acc.at[dstv.at[ci]],
                                  ssem[b]).wait()

        _gather(0, 0)
        _gather(1, 1)

        def _round(i, _):
            for b in range(NBUF):
                ci = i * NBUF + b
                _gather_wait(ci, b)

                base = ci * CH

                def _scale(g, _):
                    wg = wv[pl.ds(base + g * L, L)]
                    for j in range(L):
                        wbc = _splat(wg, j)
                        k = g * L + j
                        for db in range(HD // L):
                            sl = pl.ds(db * L, L)
                            rows[b][k, sl] = rows[b][k, sl] * wbc
                    return 0
                lax.fori_loop(0, CH // L, _scale, 0)

                pltpu.async_copy(rows[b], acc.at[dstv.at[ci]], ssem[b],
                                 add=True)

                b2 = (b + 2) % NBUF

                @pl.when(ci >= 1)
                def _():
                    _scatter_wait(ci - 1, b2)

                @pl.when(ci + 2 < NCH)
                def _():
                    _gather(ci + 2, b2)
            return 0
        lax.fori_loop(0, NCH // NBUF, _round, 0)

        _scatter_wait(NCH - 1, (NCH - 1) % NBUF)

    @pl.when(c == 0)
    def _():
        _edge_loop(y_hbm.at[0])

    @pl.when(c == 1)
    def _():
        _edge_loop(y_hbm.at[1])

    plsc.subcore_barrier()

    rsl = pl.ds(s * RPW, RPW)

    @pl.when(c == 0)
    def _():
        pltpu.sync_copy(acc.at[rsl], out_hbm.at[0].at[rsl])

    @pl.when(c == 1)
    def _():
        pltpu.sync_copy(acc.at[rsl], out_hbm.at[1].at[rsl])


def _sc_scatter(y3, src2, rel2, dst3, w2, zrows):
    mesh = plsc.VectorSubcoreMesh(core_axis_name="c", subcore_axis_name="s")
    f = pl.kernel(
        _sc_body,
        out_type=jax.ShapeDtypeStruct((NC, N, HD), jnp.float32),
        mesh=mesh,
        compiler_params=pltpu.CompilerParams(
            needs_layout_passes=False,
            use_tc_tiling_on_sc=False,
        ),
        scratch_types=(
            pltpu.VMEM((EPTP,), jnp.int32),
            pltpu.VMEM((RC,), jnp.int32),
            pltpu.VMEM((NCH, CH), jnp.int32),
            pltpu.VMEM((EPTP,), jnp.float32),
            pltpu.VMEM((CH, HD), jnp.float32),
            pltpu.VMEM((CH, HD), jnp.float32),
            pltpu.VMEM((CH, HD), jnp.float32),
            pltpu.VMEM_SHARED((N, HD), jnp.float32),
            pltpu.SemaphoreType.DMA,
            pltpu.SemaphoreType.DMA,
            pltpu.SemaphoreType.DMA,
            pltpu.SemaphoreType.DMA,
            pltpu.SemaphoreType.DMA,
            pltpu.SemaphoreType.DMA,
        ),
    )
    return f(y3, src2, rel2, dst3, w2, zrows)


def _proj_body(h_ref, m_ref, wsT_ref, b_ref, y_ref, s_ref):
    h = h_ref[...]
    y_ref[0] = jnp.dot(h, m_ref[0], preferred_element_type=jnp.float32)
    y_ref[1] = jnp.dot(h, m_ref[1], preferred_element_type=jnp.float32)
    s_ref[...] = (jnp.dot(h, wsT_ref[...], preferred_element_type=jnp.float32)
                  + b_ref[...])


def _proj(h, m2, wsT, b):
    bm = 1000
    return pl.pallas_call(
        _proj_body,
        grid=(N // bm,),
        in_specs=[pl.BlockSpec((bm, D), lambda i: (i, 0)),
                  pl.BlockSpec((NC, D, R * HD), lambda i: (0, 0, 0)),
                  pl.BlockSpec((D, D), lambda i: (0, 0)),
                  pl.BlockSpec((1, D), lambda i: (0, 0))],
        out_specs=[pl.BlockSpec((NC, bm, R * HD), lambda i: (0, i, 0)),
                   pl.BlockSpec((bm, D), lambda i: (i, 0))],
        out_shape=[jax.ShapeDtypeStruct((NC, N, R * HD), jnp.float32),
                   jax.ShapeDtypeStruct((N, D), jnp.float32)],
    )(h, m2, wsT, b)


def _comb_body(plo_ref, phi_ref, s_ref, o_ref):
    p = jnp.concatenate([plo_ref[0], phi_ref[0]], axis=1)
    o_ref[...] = jnp.maximum(p + s_ref[...], 0.0)


def _combine(p, s):
    bm = 1000
    return pl.pallas_call(
        _comb_body,
        grid=(N // bm,),
        in_specs=[pl.BlockSpec((1, bm, HD), lambda i: (0, i, 0)),
                  pl.BlockSpec((1, bm, HD), lambda i: (1, i, 0)),
                  pl.BlockSpec((bm, D), lambda i: (i, 0))],
        out_specs=pl.BlockSpec((bm, D), lambda i: (i, 0)),
        out_shape=jax.ShapeDtypeStruct((N, D), jnp.float32),
    )(p, p, s)


def kernel(x, edge_index, edge_relation, edge_weight,
           W_lin0, b_lin0, W_self0, b_self0,
           W_lin1, b_lin1, W_self1, b_self1):
    pad = ((0, 0), (0, EPTP - EPT))
    src2 = jnp.pad(edge_index[0].astype(jnp.int32).reshape(NS, EPT), pad)
    rel2 = jnp.pad(edge_relation.astype(jnp.int32).reshape(NS, EPT), pad)
    dst3 = jnp.pad(edge_index[1].astype(jnp.int32).reshape(NS, EPT),
                   pad).reshape(NS, NCH, CH)
    w2 = jnp.pad(edge_weight.astype(jnp.float32).reshape(NS, EPT), pad)
    zrows = jnp.zeros((RPW, HD), jnp.float32)

    def mk_m(W_lin):
        m3 = W_lin.reshape(D, R, D).transpose(2, 1, 0)
        return jnp.stack([m3[:, :, :HD].reshape(D, R * HD),
                          m3[:, :, HD:].reshape(D, R * HD)])

    m0, m1 = mk_m(W_lin0), mk_m(W_lin1)
    b0 = (b_lin0 + b_self0).reshape(1, D)
    b1 = (b_lin1 + b_self1).reshape(1, D)

    y0, s0 = _proj(x, m0, W_self0.T, b0)
    p0 = _sc_scatter(y0.reshape(NC, N * R, HD), src2, rel2, dst3, w2, zrows)
    h = _combine(p0, s0)

    y1, s1 = _proj(h, m1, W_self1.T, b1)
    p1 = _sc_scatter(y1.reshape(NC, N * R, HD), src2, rel2, dst3, w2, zrows)
    return _combine(p1, s1)


# speedup vs baseline: 1.6120x; 1.6120x over previous
import jax
import jax.numpy as jnp
from jax import lax

_SPLAT_DNUMS = lax.GatherDimensionNumbers(
    offset_dims=(), collapsed_slice_dims=(0,), start_index_map=(0,))


def _splat(vec, j):
    idx = jnp.full((16, 1), j, jnp.int32)
    return lax.gather(vec, idx, _SPLAT_DNUMS, (1,),
                      mode=lax.GatherScatterMode.PROMISE_IN_BOUNDS)
from jax.experimental import pallas as pl
from jax.experimental.pallas import tpu as pltpu
from jax.experimental.pallas import tpu_sc as plsc

N = 10000
E = 320000
R = 7
D = 128
HD = D // 2

NC = 2
NS = 16
EPT = E // NS
CH = 128
NCH = 159
EPTP = NCH * CH
NBUF = 3
RC = EPTP // 6
RPW = N // NS
L = 16


def _sc_body(y_hbm, src_hbm, rel_hbm, dst_hbm, w_hbm, z_hbm, out_hbm,
             srcv, relv, dstv, wv, rows0, rows1, rows2, acc,
             g0, g1, g2, s0, s1, s2):
    c = lax.axis_index("c")
    s = lax.axis_index("s")
    rows = (rows0, rows1, rows2)
    gsem = (g0, g1, g2)
    ssem = (s0, s1, s2)

    pltpu.sync_copy(z_hbm, acc.at[pl.ds(s * RPW, RPW)])

    pltpu.sync_copy(src_hbm.at[s], srcv)
    pltpu.sync_copy(dst_hbm.at[s], dstv)
    pltpu.sync_copy(w_hbm.at[s], wv)

    for seg in range(EPTP // RC):
        pltpu.sync_copy(rel_hbm.at[s].at[pl.ds(seg * RC, RC)], relv)

        def _gidx(t, _):
            gl = pl.ds(seg * RC + t * L, L)
            srcv[gl] = srcv[gl] * R + relv[pl.ds(t * L, L)]
            return 0
        lax.fori_loop(0, RC // L, _gidx, 0)

    plsc.subcore_barrier()

    def _edge_loop(y_ref):
        def _gather(ci, b):
            pltpu.async_copy(y_ref.at[srcv.at[pl.ds(ci * CH, CH)]],
                             rows[b], gsem[b])

        def _gather_wait(ci, b):
            pltpu.make_async_copy(y_ref.at[srcv.at[pl.ds(ci * CH, CH)]],
                                  rows[b], gsem[b]).wait()

        def _scatter_wait(ci, b):
            pltpu.make_async_copy(rows[b], acc.at[dstv.at[ci]],
                                  ssem[b]).wait()

        _gather(0, 0)
        _gather(1, 1)

        def _round(i, _):
            for b in range(NBUF):
                ci = i * NBUF + b
                _gather_wait(ci, b)

                base = ci * CH

                def _scale(k):
                    wbc = plsc.load_gather(
                        wv, [jnp.full((L,), base + k, jnp.int32)])
                    for db in range(HD // L):
                        sl = pl.ds(db * L, L)
                        rows[b][k, sl] = rows[b][k, sl] * wbc
                plsc.parallel_loop(0, CH, 1, unroll=4)(_scale)

                pltpu.async_copy(rows[b], acc.at[dstv.at[ci]], ssem[b],
                                 add=True)

                b2 = (b + 2) % NBUF

                @pl.when(ci >= 1)
                def _():
                    _scatter_wait(ci - 1, b2)

                @pl.when(ci + 2 < NCH)
                def _():
                    _gather(ci + 2, b2)
            return 0
        lax.fori_loop(0, NCH // NBUF, _round, 0)

        _scatter_wait(NCH - 1, (NCH - 1) % NBUF)

    @pl.when(c == 0)
    def _():
        _edge_loop(y_hbm.at[0])

    @pl.when(c == 1)
    def _():
        _edge_loop(y_hbm.at[1])

    plsc.subcore_barrier()

    rsl = pl.ds(s * RPW, RPW)

    @pl.when(c == 0)
    def _():
        pltpu.sync_copy(acc.at[rsl], out_hbm.at[0].at[rsl])

    @pl.when(c == 1)
    def _():
        pltpu.sync_copy(acc.at[rsl], out_hbm.at[1].at[rsl])


def _sc_scatter(y3, src2, rel2, dst3, w2, zrows):
    mesh = plsc.VectorSubcoreMesh(core_axis_name="c", subcore_axis_name="s")
    f = pl.kernel(
        _sc_body,
        out_type=jax.ShapeDtypeStruct((NC, N, HD), jnp.float32),
        mesh=mesh,
        compiler_params=pltpu.CompilerParams(
            needs_layout_passes=False,
            use_tc_tiling_on_sc=False,
        ),
        scratch_types=(
            pltpu.VMEM((EPTP,), jnp.int32),
            pltpu.VMEM((RC,), jnp.int32),
            pltpu.VMEM((NCH, CH), jnp.int32),
            pltpu.VMEM((EPTP,), jnp.float32),
            pltpu.VMEM((CH, HD), jnp.float32),
            pltpu.VMEM((CH, HD), jnp.float32),
            pltpu.VMEM((CH, HD), jnp.float32),
            pltpu.VMEM_SHARED((N, HD), jnp.float32),
            pltpu.SemaphoreType.DMA,
            pltpu.SemaphoreType.DMA,
            pltpu.SemaphoreType.DMA,
            pltpu.SemaphoreType.DMA,
            pltpu.SemaphoreType.DMA,
            pltpu.SemaphoreType.DMA,
        ),
    )
    return f(y3, src2, rel2, dst3, w2, zrows)


def _proj_body(h_ref, m_ref, wsT_ref, b_ref, y_ref, s_ref):
    h = h_ref[...]
    y_ref[0] = jnp.dot(h, m_ref[0], preferred_element_type=jnp.float32)
    y_ref[1] = jnp.dot(h, m_ref[1], preferred_element_type=jnp.float32)
    s_ref[...] = (jnp.dot(h, wsT_ref[...], preferred_element_type=jnp.float32)
                  + b_ref[...])


def _proj(h, m2, wsT, b):
    bm = 1000
    return pl.pallas_call(
        _proj_body,
        grid=(N // bm,),
        in_specs=[pl.BlockSpec((bm, D), lambda i: (i, 0)),
                  pl.BlockSpec((NC, D, R * HD), lambda i: (0, 0, 0)),
                  pl.BlockSpec((D, D), lambda i: (0, 0)),
                  pl.BlockSpec((1, D), lambda i: (0, 0))],
        out_specs=[pl.BlockSpec((NC, bm, R * HD), lambda i: (0, i, 0)),
                   pl.BlockSpec((bm, D), lambda i: (i, 0))],
        out_shape=[jax.ShapeDtypeStruct((NC, N, R * HD), jnp.float32),
                   jax.ShapeDtypeStruct((N, D), jnp.float32)],
    )(h, m2, wsT, b)


def _comb_body(plo_ref, phi_ref, s_ref, o_ref):
    p = jnp.concatenate([plo_ref[0], phi_ref[0]], axis=1)
    o_ref[...] = jnp.maximum(p + s_ref[...], 0.0)


def _combine(p, s):
    bm = 1000
    return pl.pallas_call(
        _comb_body,
        grid=(N // bm,),
        in_specs=[pl.BlockSpec((1, bm, HD), lambda i: (0, i, 0)),
                  pl.BlockSpec((1, bm, HD), lambda i: (1, i, 0)),
                  pl.BlockSpec((bm, D), lambda i: (i, 0))],
        out_specs=pl.BlockSpec((bm, D), lambda i: (i, 0)),
        out_shape=jax.ShapeDtypeStruct((N, D), jnp.float32),
    )(p, p, s)


def kernel(x, edge_index, edge_relation, edge_weight,
           W_lin0, b_lin0, W_self0, b_self0,
           W_lin1, b_lin1, W_self1, b_self1):
    pad = ((0, 0), (0, EPTP - EPT))
    src2 = jnp.pad(edge_index[0].astype(jnp.int32).reshape(NS, EPT), pad)
    rel2 = jnp.pad(edge_relation.astype(jnp.int32).reshape(NS, EPT), pad)
    dst3 = jnp.pad(edge_index[1].astype(jnp.int32).reshape(NS, EPT),
                   pad).reshape(NS, NCH, CH)
    w2 = jnp.pad(edge_weight.astype(jnp.float32).reshape(NS, EPT), pad)
    zrows = jnp.zeros((RPW, HD), jnp.float32)

    def mk_m(W_lin):
        m3 = W_lin.reshape(D, R, D).transpose(2, 1, 0)
        return jnp.stack([m3[:, :, :HD].reshape(D, R * HD),
                          m3[:, :, HD:].reshape(D, R * HD)])

    m0, m1 = mk_m(W_lin0), mk_m(W_lin1)
    b0 = (b_lin0 + b_self0).reshape(1, D)
    b1 = (b_lin1 + b_self1).reshape(1, D)

    y0, s0 = _proj(x, m0, W_self0.T, b0)
    p0 = _sc_scatter(y0.reshape(NC, N * R, HD), src2, rel2, dst3, w2, zrows)
    h = _combine(p0, s0)

    y1, s1 = _proj(h, m1, W_self1.T, b1)
    p1 = _sc_scatter(y1.reshape(NC, N * R, HD), src2, rel2, dst3, w2, zrows)
    return _combine(p1, s1)


# speedup vs baseline: 1.6302x; 1.0113x over previous
import jax
import jax.numpy as jnp
from jax import lax

_SPLAT_DNUMS = lax.GatherDimensionNumbers(
    offset_dims=(), collapsed_slice_dims=(0,), start_index_map=(0,))


def _splat(vec, j):
    idx = jnp.full((16, 1), j, jnp.int32)
    return lax.gather(vec, idx, _SPLAT_DNUMS, (1,),
                      mode=lax.GatherScatterMode.PROMISE_IN_BOUNDS)
from jax.experimental import pallas as pl
from jax.experimental.pallas import tpu as pltpu
from jax.experimental.pallas import tpu_sc as plsc

N = 10000
E = 320000
R = 7
D = 128
HD = D // 2

NC = 2
NS = 16
EPT = E // NS
CH = 128
NCH = 159
EPTP = NCH * CH
NBUF = 3
RC = EPTP // 6
RPW = N // NS
L = 16


def _sc_body(y_hbm, src_hbm, rel_hbm, dst_hbm, w_hbm, z_hbm, out_hbm,
             srcv, relv, dstv, wv, rows0, rows1, rows2, acc,
             g0, g1, g2, s0, s1, s2):
    c = lax.axis_index("c")
    s = lax.axis_index("s")
    rows = (rows0, rows1, rows2)
    gsem = (g0, g1, g2)
    ssem = (s0, s1, s2)

    pltpu.sync_copy(z_hbm, acc.at[pl.ds(s * RPW, RPW)])

    pltpu.sync_copy(src_hbm.at[s], srcv)
    pltpu.sync_copy(dst_hbm.at[s], dstv)
    pltpu.sync_copy(w_hbm.at[s], wv)

    for seg in range(EPTP // RC):
        pltpu.sync_copy(rel_hbm.at[s].at[pl.ds(seg * RC, RC)], relv)

        def _gidx(t):
            gl = pl.ds(seg * RC + t * L, L)
            srcv[gl] = srcv[gl] * R + relv[pl.ds(t * L, L)]
        plsc.parallel_loop(0, RC // L, 1, unroll=4)(_gidx)

    plsc.subcore_barrier()

    def _edge_loop(y_ref):
        def _gather(ci, b):
            pltpu.async_copy(y_ref.at[srcv.at[pl.ds(ci * CH, CH)]],
                             rows[b], gsem[b])

        def _gather_wait(ci, b):
            pltpu.make_async_copy(y_ref.at[srcv.at[pl.ds(ci * CH, CH)]],
                                  rows[b], gsem[b]).wait()

        def _scatter_wait(ci, b):
            pltpu.make_async_copy(rows[b], acc.at[dstv.at[ci]],
                                  ssem[b]).wait()

        _gather(0, 0)
        _gather(1, 1)

        def _round(i, _):
            for b in range(NBUF):
                ci = i * NBUF + b
                _gather_wait(ci, b)

                base = ci * CH

                def _scale(k):
                    wbc = plsc.load_gather(
                        wv, [jnp.full((L,), base + k, jnp.int32)])
                    for db in range(HD // L):
                        sl = pl.ds(db * L, L)
                        rows[b][k, sl] = rows[b][k, sl] * wbc
                plsc.parallel_loop(0, CH, 1, unroll=8)(_scale)

                pltpu.async_copy(rows[b], acc.at[dstv.at[ci]], ssem[b],
                                 add=True)

                b2 = (b + 2) % NBUF

                @pl.when(ci >= 1)
                def _():
                    _scatter_wait(ci - 1, b2)

                @pl.when(ci + 2 < NCH)
                def _():
                    _gather(ci + 2, b2)
            return 0
        lax.fori_loop(0, NCH // NBUF, _round, 0)

        _scatter_wait(NCH - 1, (NCH - 1) % NBUF)

    @pl.when(c == 0)
    def _():
        _edge_loop(y_hbm.at[0])

    @pl.when(c == 1)
    def _():
        _edge_loop(y_hbm.at[1])

    plsc.subcore_barrier()

    rsl = pl.ds(s * RPW, RPW)

    @pl.when(c == 0)
    def _():
        pltpu.sync_copy(acc.at[rsl], out_hbm.at[0].at[rsl])

    @pl.when(c == 1)
    def _():
        pltpu.sync_copy(acc.at[rsl], out_hbm.at[1].at[rsl])


def _sc_scatter(y3, src2, rel2, dst3, w2, zrows):
    mesh = plsc.VectorSubcoreMesh(core_axis_name="c", subcore_axis_name="s")
    f = pl.kernel(
        _sc_body,
        out_type=jax.ShapeDtypeStruct((NC, N, HD), jnp.float32),
        mesh=mesh,
        compiler_params=pltpu.CompilerParams(
            needs_layout_passes=False,
            use_tc_tiling_on_sc=False,
        ),
        scratch_types=(
            pltpu.VMEM((EPTP,), jnp.int32),
            pltpu.VMEM((RC,), jnp.int32),
            pltpu.VMEM((NCH, CH), jnp.int32),
            pltpu.VMEM((EPTP,), jnp.float32),
            pltpu.VMEM((CH, HD), jnp.float32),
            pltpu.VMEM((CH, HD), jnp.float32),
            pltpu.VMEM((CH, HD), jnp.float32),
            pltpu.VMEM_SHARED((N, HD), jnp.float32),
            pltpu.SemaphoreType.DMA,
            pltpu.SemaphoreType.DMA,
            pltpu.SemaphoreType.DMA,
            pltpu.SemaphoreType.DMA,
            pltpu.SemaphoreType.DMA,
            pltpu.SemaphoreType.DMA,
        ),
    )
    return f(y3, src2, rel2, dst3, w2, zrows)


def _proj_body(h_ref, m_ref, wsT_ref, b_ref, y_ref, s_ref):
    h = h_ref[...]
    y_ref[0] = jnp.dot(h, m_ref[0], preferred_element_type=jnp.float32)
    y_ref[1] = jnp.dot(h, m_ref[1], preferred_element_type=jnp.float32)
    s_ref[...] = (jnp.dot(h, wsT_ref[...], preferred_element_type=jnp.float32)
                  + b_ref[...])


def _proj(h, m2, wsT, b):
    bm = 1000
    return pl.pallas_call(
        _proj_body,
        grid=(N // bm,),
        in_specs=[pl.BlockSpec((bm, D), lambda i: (i, 0)),
                  pl.BlockSpec((NC, D, R * HD), lambda i: (0, 0, 0)),
                  pl.BlockSpec((D, D), lambda i: (0, 0)),
                  pl.BlockSpec((1, D), lambda i: (0, 0))],
        out_specs=[pl.BlockSpec((NC, bm, R * HD), lambda i: (0, i, 0)),
                   pl.BlockSpec((bm, D), lambda i: (i, 0))],
        out_shape=[jax.ShapeDtypeStruct((NC, N, R * HD), jnp.float32),
                   jax.ShapeDtypeStruct((N, D), jnp.float32)],
    )(h, m2, wsT, b)


def _comb_body(plo_ref, phi_ref, s_ref, o_ref):
    p = jnp.concatenate([plo_ref[0], phi_ref[0]], axis=1)
    o_ref[...] = jnp.maximum(p + s_ref[...], 0.0)


def _combine(p, s):
    bm = 1000
    return pl.pallas_call(
        _comb_body,
        grid=(N // bm,),
        in_specs=[pl.BlockSpec((1, bm, HD), lambda i: (0, i, 0)),
                  pl.BlockSpec((1, bm, HD), lambda i: (1, i, 0)),
                  pl.BlockSpec((bm, D), lambda i: (i, 0))],
        out_specs=pl.BlockSpec((bm, D), lambda i: (i, 0)),
        out_shape=jax.ShapeDtypeStruct((N, D), jnp.float32),
    )(p, p, s)


def kernel(x, edge_index, edge_relation, edge_weight,
           W_lin0, b_lin0, W_self0, b_self0,
           W_lin1, b_lin1, W_self1, b_self1):
    pad = ((0, 0), (0, EPTP - EPT))
    src2 = jnp.pad(edge_index[0].astype(jnp.int32).reshape(NS, EPT), pad)
    rel2 = jnp.pad(edge_relation.astype(jnp.int32).reshape(NS, EPT), pad)
    dst3 = jnp.pad(edge_index[1].astype(jnp.int32).reshape(NS, EPT),
                   pad).reshape(NS, NCH, CH)
    w2 = jnp.pad(edge_weight.astype(jnp.float32).reshape(NS, EPT), pad)
    zrows = jnp.zeros((RPW, HD), jnp.float32)

    def mk_m(W_lin):
        m3 = W_lin.reshape(D, R, D).transpose(2, 1, 0)
        return jnp.stack([m3[:, :, :HD].reshape(D, R * HD),
                          m3[:, :, HD:].reshape(D, R * HD)])

    m0, m1 = mk_m(W_lin0), mk_m(W_lin1)
    b0 = (b_lin0 + b_self0).reshape(1, D)
    b1 = (b_lin1 + b_self1).reshape(1, D)

    y0, s0 = _proj(x, m0, W_self0.T, b0)
    p0 = _sc_scatter(y0.reshape(NC, N * R, HD), src2, rel2, dst3, w2, zrows)
    h = _combine(p0, s0)

    y1, s1 = _proj(h, m1, W_self1.T, b1)
    p1 = _sc_scatter(y1.reshape(NC, N * R, HD), src2, rel2, dst3, w2, zrows)
    return _combine(p1, s1)
